# Initial kernel scaffold; baseline (speedup 1.0000x reference)
import jax
import jax.numpy as jnp
from jax.experimental import pallas as pl


def kernel(tokens, table, W, b):
    raise NotImplementedError("write your pallas kernel here")


# speedup vs baseline: 2.7794x; 2.7794x over previous
import functools

import jax
import jax.numpy as jnp
from jax import lax
from jax.experimental import pallas as pl
from jax.experimental.pallas import tpu as pltpu
from jax.experimental.pallas import tpu_sc as plsc

BATCH = 16384
BAG_LEN = 50
D = 64

NC = 2
NS = 16
NW = NC * NS
BAGS_PER_W = BATCH // NW
STEP_BAGS = 2
ROWS_PER_STEP = STEP_BAGS * BAG_LEN
NSTEP = BAGS_PER_W // STEP_BAGS
NBUF = 4
UNROLL = 5


def _bag_sum_sc(tokens2d, table):
    mesh = plsc.VectorSubcoreMesh(core_axis_name="c", subcore_axis_name="s")

    @functools.partial(
        pl.kernel,
        out_type=jax.ShapeDtypeStruct((BATCH, D), jnp.float32),
        mesh=mesh,
        scratch_types=[
            pltpu.VMEM((NSTEP, ROWS_PER_STEP), jnp.int32),
            pltpu.VMEM((NBUF, ROWS_PER_STEP, D), jnp.float32),
            pltpu.VMEM((BAGS_PER_W, D), jnp.float32),
            pltpu.SemaphoreType.DMA,
        ],
        compiler_params=pltpu.CompilerParams(use_tc_tiling_on_sc=False),
    )
    def kern(tokens_hbm, table_hbm, out_hbm, idx_v, ring_v, pooled_v, sem):
        wid = lax.axis_index("s") * NC + lax.axis_index("c")
        row_base = wid * NSTEP

        pltpu.sync_copy(tokens_hbm.at[pl.ds(row_base, NSTEP)], idx_v)

        for s in range(NBUF):
            pltpu.async_copy(table_hbm.at[idx_v.at[s]], ring_v.at[s], sem)

        def accumulate(slot, bag, j):
            def body(i, carry):
                accs = list(carry)
                for u in range(UNROLL):
                    r = bag * BAG_LEN + i * UNROLL + u
                    for q in range(D // 16):
                        accs[q] = accs[q] + ring_v[slot, r, pl.ds(q * 16, 16)]
                return tuple(accs)

            zeros = tuple(jnp.zeros((16,), jnp.float32) for _ in range(D // 16))
            accs = lax.fori_loop(0, BAG_LEN // UNROLL, body, zeros)
            for q in range(D // 16):
                pooled_v[j * STEP_BAGS + bag, pl.ds(q * 16, 16)] = accs[q]

        @pl.loop(0, NSTEP, step=NBUF)
        def _steps(j0):
            for s in range(NBUF):
                j = j0 + s
                pltpu.make_async_copy(
                    table_hbm.at[idx_v.at[0]], ring_v.at[s], sem
                ).wait()
                for bag in range(STEP_BAGS):
                    accumulate(s, bag, j)
                nj = j + NBUF

                @pl.when(nj < NSTEP)
                def _():
                    pltpu.async_copy(
                        table_hbm.at[idx_v.at[nj]], ring_v.at[s], sem
                    )

        pltpu.sync_copy(pooled_v, out_hbm.at[pl.ds(wid * BAGS_PER_W, BAGS_PER_W)])

    return kern(tokens2d, table)


def _project_tc(pooled_sum, W, b2d):
    BLK = 2048

    def body(p_ref, w_ref, b_ref, o_ref):
        x = p_ref[...] * (1.0 / BAG_LEN)
        o_ref[...] = (
            lax.dot_general(
                x, w_ref[...], (((1,), (1,)), ((), ())),
                preferred_element_type=jnp.float32,
            )
            + b_ref[...]
        )

    return pl.pallas_call(
        body,
        out_shape=jax.ShapeDtypeStruct((BATCH, D), jnp.float32),
        grid=(BATCH // BLK,),
        in_specs=[
            pl.BlockSpec((BLK, D), lambda i: (i, 0)),
            pl.BlockSpec((D, D), lambda i: (0, 0)),
            pl.BlockSpec((1, D), lambda i: (0, 0)),
        ],
        out_specs=pl.BlockSpec((BLK, D), lambda i: (i, 0)),
    )(pooled_sum, W, b2d)


@jax.jit
def kernel(tokens, table, W, b):
    tokens2d = tokens.astype(jnp.int32).reshape(NW * NSTEP, ROWS_PER_STEP)
    pooled_sum = _bag_sum_sc(tokens2d, table)
    return _project_tc(pooled_sum, W, b.reshape(1, D))


# speedup vs baseline: 3.3863x; 1.2184x over previous
import functools

import jax
import jax.numpy as jnp
from jax import lax
from jax.experimental import pallas as pl
from jax.experimental.pallas import tpu as pltpu
from jax.experimental.pallas import tpu_sc as plsc

BATCH = 16384
BAG_LEN = 50
D = 64
VOCAB = 1000000

NC = 2
NS = 16
NW = NC * NS
BAGS_PER_W = BATCH // NW
STEP_BAGS = 2
ROWS_PER_STEP = STEP_BAGS * BAG_LEN
NSTEP = BAGS_PER_W // STEP_BAGS
NBUF = 4
UNROLL = 5


def _bag_sum_sc(tokens2d, table):
    mesh = plsc.VectorSubcoreMesh(core_axis_name="c", subcore_axis_name="s")

    @functools.partial(
        pl.kernel,
        out_type=jax.ShapeDtypeStruct((BATCH, D), jnp.float32),
        mesh=mesh,
        scratch_types=[
            pltpu.VMEM((NSTEP, ROWS_PER_STEP), jnp.int32),
            pltpu.VMEM((NBUF, ROWS_PER_STEP, D), jnp.float32),
            pltpu.VMEM((BAGS_PER_W, D), jnp.float32),
            pltpu.SemaphoreType.DMA,
        ],
        compiler_params=pltpu.CompilerParams(use_tc_tiling_on_sc=False),
    )
    def kern(tokens_hbm, table_hbm, out_hbm, idx_v, ring_v, pooled_v, sem):
        wid = lax.axis_index("s") * NC + lax.axis_index("c")
        row_base = wid * NSTEP

        pltpu.sync_copy(tokens_hbm.at[pl.ds(row_base, NSTEP)], idx_v)

        for s in range(NBUF):
            pltpu.async_copy(table_hbm.at[idx_v.at[s]], ring_v.at[s], sem)

        def accumulate(slot, bag, j):
            def body(i, carry):
                accs = list(carry)
                for u in range(UNROLL):
                    r = bag * BAG_LEN + i * UNROLL + u
                    for q in range(D // 16):
                        accs[q] = accs[q] + ring_v[slot, r, pl.ds(q * 16, 16)]
                return tuple(accs)

            zeros = tuple(jnp.zeros((16,), jnp.float32) for _ in range(D // 16))
            accs = lax.fori_loop(0, BAG_LEN // UNROLL, body, zeros)
            for q in range(D // 16):
                pooled_v[j * STEP_BAGS + bag, pl.ds(q * 16, 16)] = accs[q]

        @pl.loop(0, NSTEP, step=NBUF)
        def _steps(j0):
            for s in range(NBUF):
                j = j0 + s
                pltpu.make_async_copy(
                    table_hbm.at[idx_v.at[0]], ring_v.at[s], sem
                ).wait()
                for bag in range(STEP_BAGS):
                    accumulate(s, bag, j)
                nj = j + NBUF

                @pl.when(nj < NSTEP)
                def _():
                    pltpu.async_copy(
                        table_hbm.at[idx_v.at[nj]], ring_v.at[s], sem
                    )

        pltpu.sync_copy(pooled_v, out_hbm.at[pl.ds(wid * BAGS_PER_W, BAGS_PER_W)])

    return kern(tokens2d, table)


def _project_tc(pooled_sum, W, b2d):
    BLK = 2048

    def body(p_ref, w_ref, b_ref, o_ref):
        x = p_ref[...] * (1.0 / BAG_LEN)
        o_ref[...] = (
            lax.dot_general(
                x, w_ref[...], (((1,), (1,)), ((), ())),
                preferred_element_type=jnp.float32,
            )
            + b_ref[...]
        )

    return pl.pallas_call(
        body,
        out_shape=jax.ShapeDtypeStruct((BATCH, D), jnp.float32),
        grid=(BATCH // BLK,),
        in_specs=[
            pl.BlockSpec((BLK, D), lambda i: (i, 0)),
            pl.BlockSpec((D, D), lambda i: (0, 0)),
            pl.BlockSpec((1, D), lambda i: (0, 0)),
        ],
        out_specs=pl.BlockSpec((BLK, D), lambda i: (i, 0)),
    )(pooled_sum, W, b2d)


REPACK_CHUNK = 2048
NBLK = (VOCAB + REPACK_CHUNK - 1) // REPACK_CHUNK
VOCAB2 = NBLK * REPACK_CHUNK


def _repack_tc(table_T):
    def body(x_ref, o_ref):
        xt = jnp.transpose(x_ref[...])
        lo = xt[: REPACK_CHUNK // 2]
        hi = xt[REPACK_CHUNK // 2 :]
        o_ref[...] = jnp.concatenate([lo, hi], axis=1)

    return pl.pallas_call(
        body,
        out_shape=jax.ShapeDtypeStruct((VOCAB2 // 2, 2 * D), jnp.float32),
        grid=(NBLK,),
        in_specs=[pl.BlockSpec((D, REPACK_CHUNK), lambda i: (0, i))],
        out_specs=pl.BlockSpec((REPACK_CHUNK // 2, 2 * D), lambda i: (i, 0)),
    )(table_T)


@jax.jit
def kernel(tokens, table, W, b):
    table128 = _repack_tc(table.T)
    table128 = jax.lax.optimization_barrier(table128)
    table_lin = table128.reshape(VOCAB2, D)
    t = tokens.astype(jnp.int32)
    rows = 2048 * (t // 2048) + (t % 1024) * 2 + (t % 2048) // 1024
    tokens2d = rows.reshape(NW * NSTEP, ROWS_PER_STEP)
    pooled_sum = _bag_sum_sc(tokens2d, table_lin)
    return _project_tc(pooled_sum, W, b.reshape(1, D))
